# baseline (speedup 1.0000x reference)
import functools

import jax
import jax.numpy as jnp
from jax import lax
from jax.experimental import pallas as pl
from jax.experimental.pallas import tpu as pltpu
from jax.experimental.pallas import tpu_sc as plsc

VOCAB = 100000
EMB_DIM = 128
BATCH = 4096
SEQ_LEN = 50

NC = 1
NS = 16
NW = NC * NS
NBPW = BATCH // NW
NBUF = 4
assert NBPW % NBUF == 0

_mesh = plsc.VectorSubcoreMesh(
    core_axis_name="c", subcore_axis_name="s", num_cores=NC
)


@functools.partial(
    pl.kernel,
    mesh=_mesh,
    out_type=jax.ShapeDtypeStruct((BATCH, SEQ_LEN, EMB_DIM), jnp.float32),
    scratch_types=(
        [pltpu.VMEM((NBPW, SEQ_LEN), jnp.int32)]
        + [pltpu.VMEM((SEQ_LEN, EMB_DIM), jnp.float32)] * NBUF
        + [pltpu.SemaphoreType.DMA] * NBUF
    ),
    compiler_params=pltpu.CompilerParams(use_tc_tiling_on_sc=True),
)
def _sc_gather(idx_hbm, table_hbm, out_hbm, idx_v, *bufs):
    rows = bufs[:NBUF]
    sems = bufs[NBUF:]
    wid = lax.axis_index("s") * NC + lax.axis_index("c")
    base = wid * NBPW

    pltpu.sync_copy(idx_hbm.at[pl.ds(base, NBPW)], idx_v)

    for b in range(NBUF):
        pltpu.async_copy(table_hbm.at[idx_v.at[b]], rows[b], sems[b])

    def body(i, carry):
        for b in range(NBUF):
            j = i * NBUF + b
            pltpu.make_async_copy(
                table_hbm.at[idx_v.at[j]], rows[b], sems[b]
            ).wait()
            pltpu.sync_copy(rows[b], out_hbm.at[base + j])

            @pl.when(j + NBUF < NBPW)
            def _():
                pltpu.async_copy(
                    table_hbm.at[idx_v.at[j + NBUF]], rows[b], sems[b]
                )

        return carry

    lax.fori_loop(0, NBPW // NBUF, body, 0)


def kernel(word_inputs, word_seq_lengths, embedding_weight):
    del word_seq_lengths
    return _sc_gather(word_inputs, embedding_weight)


# speedup vs baseline: 2.0966x; 2.0966x over previous
import functools

import jax
import jax.numpy as jnp
from jax import lax
from jax.experimental import pallas as pl
from jax.experimental.pallas import tpu as pltpu
from jax.experimental.pallas import tpu_sc as plsc

VOCAB = 100000
EMB_DIM = 128
BATCH = 4096
SEQ_LEN = 50

NC = 2
NS = 16
NW = NC * NS
CHUNK = BATCH // NW
NBUF = 5
assert SEQ_LEN % NBUF == 0

_mesh = plsc.VectorSubcoreMesh(
    core_axis_name="c", subcore_axis_name="s", num_cores=NC
)


@functools.partial(
    pl.kernel,
    mesh=_mesh,
    out_type=jax.ShapeDtypeStruct((SEQ_LEN, BATCH, EMB_DIM), jnp.float32),
    scratch_types=(
        [pltpu.VMEM((SEQ_LEN, CHUNK), jnp.int32)]
        + [pltpu.VMEM((CHUNK, EMB_DIM), jnp.float32)] * NBUF
        + [pltpu.SemaphoreType.DMA] * NBUF
    ),
    compiler_params=pltpu.CompilerParams(use_tc_tiling_on_sc=True),
)
def _sc_gather(idx_hbm, table_hbm, out_hbm, idx_v, *bufs):
    rows = bufs[:NBUF]
    sems = bufs[NBUF:]
    wid = lax.axis_index("s") * NC + lax.axis_index("c")
    base = wid * CHUNK

    pltpu.sync_copy(idx_hbm.at[:, pl.ds(base, CHUNK)], idx_v)

    for b in range(NBUF):
        pltpu.async_copy(table_hbm.at[idx_v.at[b]], rows[b], sems[b])

    def body(i, carry):
        for b in range(NBUF):
            s = i * NBUF + b
            pltpu.make_async_copy(
                table_hbm.at[idx_v.at[s]], rows[b], sems[b]
            ).wait()
            pltpu.sync_copy(rows[b], out_hbm.at[s, pl.ds(base, CHUNK)])

            @pl.when(s + NBUF < SEQ_LEN)
            def _():
                pltpu.async_copy(
                    table_hbm.at[idx_v.at[s + NBUF]], rows[b], sems[b]
                )

        return carry

    lax.fori_loop(0, SEQ_LEN // NBUF, body, 0)


def kernel(word_inputs, word_seq_lengths, embedding_weight):
    del word_seq_lengths
    out = _sc_gather(word_inputs.T, embedding_weight)
    return out.transpose(1, 0, 2)


# speedup vs baseline: 2.0991x; 1.0012x over previous
import functools

import jax
import jax.numpy as jnp
from jax import lax
from jax.experimental import pallas as pl
from jax.experimental.pallas import tpu as pltpu
from jax.experimental.pallas import tpu_sc as plsc

VOCAB = 100000
EMB_DIM = 128
BATCH = 4096
SEQ_LEN = 50

NC = 2
NS = 16
NW = NC * NS
STRIPE = BATCH // NW
HALF = STRIPE // 2
NCHUNK = SEQ_LEN * 2
LOOK = 5
NBUF = 2 * LOOK

_mesh = plsc.VectorSubcoreMesh(
    core_axis_name="c", subcore_axis_name="s", num_cores=NC
)


@functools.partial(
    pl.kernel,
    mesh=_mesh,
    out_type=jax.ShapeDtypeStruct((SEQ_LEN, BATCH, EMB_DIM), jnp.float32),
    scratch_types=(
        [pltpu.VMEM((SEQ_LEN, STRIPE), jnp.int32)]
        + [pltpu.VMEM((HALF, EMB_DIM), jnp.float32)] * NBUF
        + [pltpu.SemaphoreType.DMA] * NBUF
        + [pltpu.SemaphoreType.DMA] * NBUF
    ),
    compiler_params=pltpu.CompilerParams(use_tc_tiling_on_sc=True),
)
def _sc_gather(idx_hbm, table_hbm, out_hbm, idx_v, *bufs):
    rows = bufs[:NBUF]
    gsem = bufs[NBUF : 2 * NBUF]
    wsem = bufs[2 * NBUF :]
    wid = lax.axis_index("s") * NC + lax.axis_index("c")
    base = wid * STRIPE

    def idx_slice(j):
        return idx_v.at[j // 2, pl.ds((j % 2) * HALF, HALF)]

    def out_slice(j):
        return out_hbm.at[j // 2, pl.ds(base + (j % 2) * HALF, HALF)]

    pltpu.sync_copy(idx_hbm.at[:, pl.ds(base, STRIPE)], idx_v)

    for b in range(LOOK):
        pltpu.async_copy(table_hbm.at[idx_slice(b)], rows[b], gsem[b])

    def body(i, carry):
        for k in range(NBUF):
            j = i * NBUF + k
            b = k
            bn = (k + LOOK) % NBUF
            pltpu.make_async_copy(
                table_hbm.at[idx_slice(j)], rows[b], gsem[b]
            ).wait()
            pltpu.async_copy(rows[b], out_slice(j), wsem[b])

            @pl.when(jnp.logical_and(j + LOOK < NCHUNK, j >= LOOK))
            def _():
                pltpu.make_async_copy(
                    rows[bn], out_slice(j - LOOK), wsem[bn]
                ).wait()

            @pl.when(j + LOOK < NCHUNK)
            def _():
                pltpu.async_copy(
                    table_hbm.at[idx_slice(j + LOOK)], rows[bn], gsem[bn]
                )

        return carry

    lax.fori_loop(0, NCHUNK // NBUF, body, 0)

    for k in range(NBUF):
        j = NCHUNK - NBUF + k
        pltpu.make_async_copy(rows[k], out_slice(j), wsem[k]).wait()


def kernel(word_inputs, word_seq_lengths, embedding_weight):
    del word_seq_lengths
    out = _sc_gather(word_inputs.T, embedding_weight)
    return out.transpose(1, 0, 2)


# speedup vs baseline: 2.1023x; 1.0015x over previous
import functools

import jax
import jax.numpy as jnp
from jax import lax
from jax.experimental import pallas as pl
from jax.experimental.pallas import tpu as pltpu
from jax.experimental.pallas import tpu_sc as plsc

VOCAB = 100000
EMB_DIM = 128
BATCH = 4096
SEQ_LEN = 50

NC = 2
NS = 16
NW = NC * NS
CHUNK = BATCH // NW
NBUF = 5
assert SEQ_LEN % NBUF == 0

_mesh = plsc.VectorSubcoreMesh(
    core_axis_name="c", subcore_axis_name="s", num_cores=NC
)


@functools.partial(
    pl.kernel,
    mesh=_mesh,
    out_type=jax.ShapeDtypeStruct((SEQ_LEN, BATCH, EMB_DIM), jnp.float32),
    scratch_types=(
        [pltpu.VMEM((SEQ_LEN, CHUNK), jnp.int32)]
        + [pltpu.VMEM((CHUNK, EMB_DIM), jnp.float32)] * NBUF
        + [pltpu.SemaphoreType.DMA] * NBUF
    ),
    compiler_params=pltpu.CompilerParams(use_tc_tiling_on_sc=True),
)
def _sc_gather(idx_hbm, table_hbm, out_hbm, idx_v, *bufs):
    rows = bufs[:NBUF]
    sems = bufs[NBUF:]
    wid = lax.axis_index("s") * NC + lax.axis_index("c")
    base = wid * CHUNK

    pltpu.sync_copy(idx_hbm.at[:, pl.ds(base, CHUNK)], idx_v)

    for b in range(NBUF):
        pltpu.async_copy(table_hbm.at[idx_v.at[b]], rows[b], sems[b])

    def body(i, carry):
        for b in range(NBUF):
            s = i * NBUF + b
            pltpu.make_async_copy(
                table_hbm.at[idx_v.at[s]], rows[b], sems[b]
            ).wait()
            pltpu.sync_copy(rows[b], out_hbm.at[s, pl.ds(base, CHUNK)])

            @pl.when(s + NBUF < SEQ_LEN)
            def _():
                pltpu.async_copy(
                    table_hbm.at[idx_v.at[s + NBUF]], rows[b], sems[b]
                )

        return carry

    lax.fori_loop(0, SEQ_LEN // NBUF, body, 0)


def kernel(word_inputs, word_seq_lengths, embedding_weight):
    del word_seq_lengths
    out = _sc_gather(word_inputs.T, embedding_weight)
    return out.transpose(1, 0, 2)
